# Initial kernel scaffold; baseline (speedup 1.0000x reference)
import functools

import jax
import jax.numpy as jnp
from jax import lax
from jax.experimental import pallas as pl
from jax.experimental.pallas import tpu as pltpu
from jax.experimental.pallas import tpu_sc as plsc

_N = 16384
_M = 16384
_NNZ = 2621440
_C = 32

_NC = 2
_NS = 16
_NW = _NC * _NS
_K = 128
_PER_W = _NNZ // _NW
_CHUNKS = _PER_W // _K
_RPT = _N // _NS
_ZR = 128


def _sc_spmm(col, row, val, x0):
  mesh = plsc.VectorSubcoreMesh(core_axis_name="c", subcore_axis_name="s")

  @functools.partial(
      pl.kernel,
      out_type=jax.ShapeDtypeStruct((_NC * _N, _C), jnp.float32),
      mesh=mesh,
      scratch_types=[
          pltpu.VMEM((_K,), jnp.int32),
          pltpu.VMEM((_K,), jnp.int32),
          pltpu.VMEM((_K,), jnp.float32),
          pltpu.VMEM((_K, _C), jnp.float32),
          pltpu.VMEM((_ZR, _C), jnp.float32),
          pltpu.VMEM_SHARED((_N, _C), jnp.float32),
          pltpu.SemaphoreType.DMA,
      ],
  )
  def k(col_h, row_h, val_h, x0_h, out_h,
        col_v, row_v, val_v, rows_v, zero_v, acc, sem):
    cid = lax.axis_index("c")
    sid = lax.axis_index("s")
    wid = sid * _NC + cid

    zeros16 = jnp.zeros((16,), jnp.float32)

    @pl.loop(0, _ZR)
    def _(r):
      zero_v[r, pl.ds(0, 16)] = zeros16
      zero_v[r, pl.ds(16, 16)] = zeros16

    for b in range(_RPT // _ZR):
      pltpu.sync_copy(zero_v, acc.at[pl.ds(sid * _RPT + b * _ZR, _ZR)])
    plsc.subcore_barrier()

    base0 = wid * _PER_W

    @pl.loop(0, _CHUNKS)
    def _(i):
      off = base0 + i * _K
      pltpu.sync_copy(col_h.at[pl.ds(off, _K)], col_v)
      pltpu.sync_copy(row_h.at[pl.ds(off, _K)], row_v)
      pltpu.sync_copy(val_h.at[pl.ds(off, _K)], val_v)
      pltpu.async_copy(x0_h.at[col_v], rows_v, sem).wait()

      @pl.loop(0, _K)
      def _(j):
        v = val_v[j]
        rows_v[j, pl.ds(0, 16)] = rows_v[j, pl.ds(0, 16)] * v
        rows_v[j, pl.ds(16, 16)] = rows_v[j, pl.ds(16, 16)] * v

      pltpu.sync_copy(rows_v, acc.at[row_v], add=True)

    plsc.subcore_barrier()
    pltpu.sync_copy(acc.at[pl.ds(sid * _RPT, _RPT)],
                    out_h.at[pl.ds(cid * _N + sid * _RPT, _RPT)])

  return k(col, row, val, x0)


def _mix(partials, wbd, brow):
  bn = 2048

  def body(p_ref, w_ref, b_ref, o_ref):
    s = p_ref[0] + p_ref[1]
    zt = lax.dot_general(w_ref[:], s, (((0,), (1,)), ((), ())),
                         preferred_element_type=jnp.float32)
    o_ref[:] = (zt + b_ref[:]).reshape(2, 16, bn)

  return pl.pallas_call(
      body,
      grid=(_N // bn,),
      in_specs=[
          pl.BlockSpec((2, bn, _C), lambda i: (0, i, 0)),
          pl.BlockSpec((_C, _C), lambda i: (0, 0)),
          pl.BlockSpec((_C, 1), lambda i: (0, 0)),
      ],
      out_specs=pl.BlockSpec((2, 16, bn), lambda i: (0, 0, i)),
      out_shape=jax.ShapeDtypeStruct((2, 16, _N), jnp.float32),
  )(partials, wbd, brow)


def kernel(D_indices, D_values, x, theta, bias):
  row = D_indices[0]
  col = D_indices[1]
  x0 = jnp.transpose(x, (2, 0, 1)).reshape(_M, _C)
  partials = _sc_spmm(col, row, D_values, x0).reshape(_NC, _N, _C)
  wbd = jnp.kron(jnp.eye(2, dtype=theta.dtype), theta.T)
  brow = jnp.concatenate([bias[0, :, 0], bias[0, :, 0]]).reshape(_C, 1)
  return _mix(partials, wbd, brow)


# speedup vs baseline: 10.6095x; 10.6095x over previous
import functools

import jax
import jax.numpy as jnp
from jax import lax
from jax.experimental import pallas as pl
from jax.experimental.pallas import tpu as pltpu
from jax.experimental.pallas import tpu_sc as plsc

_N = 16384
_M = 16384
_NNZ = 2621440
_C = 32

_NC = 2
_NS = 16
_NW = _NC * _NS
_K = 128
_PER_W = _NNZ // _NW
_CHUNKS = _PER_W // _K
_RPT = _N // _NS
_ZR = 128


def _sc_spmm(col, row, val, x0):
  mesh = plsc.VectorSubcoreMesh(core_axis_name="c", subcore_axis_name="s")

  @functools.partial(
      pl.kernel,
      out_type=jax.ShapeDtypeStruct((_NC * _N, _C), jnp.float32),
      mesh=mesh,
      scratch_types=[
          pltpu.VMEM((_K,), jnp.int32),
          pltpu.VMEM((_K,), jnp.int32),
          pltpu.VMEM((_K,), jnp.float32),
          pltpu.VMEM((_K, _C), jnp.float32),
          pltpu.VMEM((_ZR, _C), jnp.float32),
          pltpu.VMEM_SHARED((_N, _C), jnp.float32),
          pltpu.SemaphoreType.DMA,
      ],
      compiler_params=pltpu.CompilerParams(use_tc_tiling_on_sc=False),
  )
  def k(col_h, row_h, val_h, x0_h, out_h,
        col_v, row_v, val_v, rows_v, zero_v, acc, sem):
    cid = lax.axis_index("c")
    sid = lax.axis_index("s")
    wid = sid * _NC + cid

    zeros16 = jnp.zeros((16,), jnp.float32)

    @pl.loop(0, _ZR)
    def _(r):
      zero_v[r, pl.ds(0, 16)] = zeros16
      zero_v[r, pl.ds(16, 16)] = zeros16

    for b in range(_RPT // _ZR):
      pltpu.sync_copy(zero_v, acc.at[pl.ds(sid * _RPT + b * _ZR, _ZR)])
    plsc.subcore_barrier()

    base0 = wid * _PER_W

    @pl.loop(0, _CHUNKS)
    def _(i):
      off = base0 + i * _K
      pltpu.sync_copy(col_h.at[pl.ds(off, _K)], col_v)
      pltpu.sync_copy(row_h.at[pl.ds(off, _K)], row_v)
      pltpu.sync_copy(val_h.at[pl.ds(off, _K)], val_v)
      pltpu.async_copy(x0_h.at[col_v], rows_v, sem).wait()

      @pl.loop(0, _K // 16)
      def _(g):
        vvec = val_v[pl.ds(g * 16, 16)]
        for j in range(16):
          v = vvec[j]
          r = g * 16 + j
          rows_v[r, pl.ds(0, 16)] = rows_v[r, pl.ds(0, 16)] * v
          rows_v[r, pl.ds(16, 16)] = rows_v[r, pl.ds(16, 16)] * v

      pltpu.sync_copy(rows_v, acc.at[row_v], add=True)

    plsc.subcore_barrier()
    pltpu.sync_copy(acc.at[pl.ds(sid * _RPT, _RPT)],
                    out_h.at[pl.ds(cid * _N + sid * _RPT, _RPT)])

  return k(col, row, val, x0)


def _mix(partials, wbd, brow):
  bn = 2048

  def body(p_ref, w_ref, b_ref, o_ref):
    s = p_ref[0] + p_ref[1]
    zt = lax.dot_general(w_ref[:], s, (((0,), (1,)), ((), ())),
                         preferred_element_type=jnp.float32)
    o_ref[:] = (zt + b_ref[:]).reshape(2, 16, bn)

  return pl.pallas_call(
      body,
      grid=(_N // bn,),
      in_specs=[
          pl.BlockSpec((2, bn, _C), lambda i: (0, i, 0)),
          pl.BlockSpec((_C, _C), lambda i: (0, 0)),
          pl.BlockSpec((_C, 1), lambda i: (0, 0)),
      ],
      out_specs=pl.BlockSpec((2, 16, bn), lambda i: (0, 0, i)),
      out_shape=jax.ShapeDtypeStruct((2, 16, _N), jnp.float32),
  )(partials, wbd, brow)


def kernel(D_indices, D_values, x, theta, bias):
  row = D_indices[0]
  col = D_indices[1]
  x0 = jnp.transpose(x, (2, 0, 1)).reshape(_M, _C)
  partials = _sc_spmm(col, row, D_values, x0).reshape(_NC, _N, _C)
  wbd = jnp.kron(jnp.eye(2, dtype=theta.dtype), theta.T)
  brow = jnp.concatenate([bias[0, :, 0], bias[0, :, 0]]).reshape(_C, 1)
  return _mix(partials, wbd, brow)


# speedup vs baseline: 25.6801x; 2.4205x over previous
import functools

import jax
import jax.numpy as jnp
from jax import lax
from jax.experimental import pallas as pl
from jax.experimental.pallas import tpu as pltpu
from jax.experimental.pallas import tpu_sc as plsc

_N = 16384
_M = 16384
_NNZ = 2621440
_C = 32

_NC = 2
_NS = 16
_NW = _NC * _NS
_K = 128
_SUB = 8
_PER_W = _NNZ // _NW
_CHUNKS = _PER_W // _K
_NB = _CHUNKS // _SUB
_RPT = _N // _NS
_ZR = 128


def _sc_spmm(col2, row2, val2, x0):
  mesh = plsc.VectorSubcoreMesh(core_axis_name="c", subcore_axis_name="s")

  @functools.partial(
      pl.kernel,
      out_type=jax.ShapeDtypeStruct((_NC * _N, _C), jnp.float32),
      mesh=mesh,
      scratch_types=[
          pltpu.VMEM((2, _SUB, _K), jnp.int32),
          pltpu.VMEM((2, _SUB, _K), jnp.int32),
          pltpu.VMEM((2, _SUB, _K), jnp.float32),
          pltpu.VMEM((2, _K, _C), jnp.float32),
          pltpu.VMEM((_ZR, _C), jnp.float32),
          pltpu.VMEM_SHARED((_N, _C), jnp.float32),
          pltpu.SemaphoreType.DMA,
          pltpu.SemaphoreType.DMA,
          pltpu.SemaphoreType.DMA,
      ],
      compiler_params=pltpu.CompilerParams(use_tc_tiling_on_sc=False),
  )
  def k(col_h, row_h, val_h, x0_h, out_h,
        colb, rowb, valb, rowsb, zero_v, acc, isem, gsem, ssem):
    cid = lax.axis_index("c")
    sid = lax.axis_index("s")
    wid = sid * _NC + cid

    zeros16 = jnp.zeros((16,), jnp.float32)

    @pl.loop(0, _ZR)
    def _(r):
      zero_v[r, pl.ds(0, 16)] = zeros16
      zero_v[r, pl.ds(16, 16)] = zeros16

    for b in range(_RPT // _ZR):
      pltpu.sync_copy(zero_v, acc.at[pl.ds(sid * _RPT + b * _ZR, _ZR)])
    plsc.subcore_barrier()

    crow0 = wid * _CHUNKS

    def stage_block(b, slot):
      pltpu.async_copy(col_h.at[pl.ds(crow0 + b * _SUB, _SUB)],
                       colb.at[slot], isem)
      pltpu.async_copy(row_h.at[pl.ds(crow0 + b * _SUB, _SUB)],
                       rowb.at[slot], isem)
      pltpu.async_copy(val_h.at[pl.ds(crow0 + b * _SUB, _SUB)],
                       valb.at[slot], isem)

    def drain_idx(slot):
      pltpu.make_async_copy(col_h.at[pl.ds(crow0, _SUB)],
                            colb.at[slot], isem).wait()
      pltpu.make_async_copy(row_h.at[pl.ds(crow0, _SUB)],
                            rowb.at[slot], isem).wait()
      pltpu.make_async_copy(val_h.at[pl.ds(crow0, _SUB)],
                            valb.at[slot], isem).wait()

    def drain_payload(sem, p):
      pltpu.make_async_copy(x0_h.at[pl.ds(0, _K)], rowsb.at[p], sem).wait()

    stage_block(0, 0)
    drain_idx(0)
    pltpu.async_copy(x0_h.at[colb.at[0, 0]], rowsb.at[0], gsem)

    @pl.loop(0, _CHUNKS)
    def _(q):
      p = lax.rem(q, 2)
      b = lax.div(q, _SUB)
      s = lax.rem(q, _SUB)
      bb = lax.rem(b, 2)

      @pl.when(jnp.logical_and(s == 0, b + 1 < _NB))
      def _():
        stage_block(b + 1, lax.rem(b + 1, 2))

      drain_payload(gsem, p)

      @pl.loop(0, _K // 16)
      def _(g):
        vvec = valb[bb, s, pl.ds(g * 16, 16)]
        for j in range(16):
          v = vvec[j]
          r = g * 16 + j
          rowsb[p, r, pl.ds(0, 16)] = rowsb[p, r, pl.ds(0, 16)] * v
          rowsb[p, r, pl.ds(16, 16)] = rowsb[p, r, pl.ds(16, 16)] * v

      @pl.when(q >= 1)
      def _():
        drain_payload(ssem, 1 - p)

      @pl.when(q + 1 < _CHUNKS)
      def _():
        q1 = q + 1
        b2 = lax.rem(lax.div(q1, _SUB), 2)
        s2 = lax.rem(q1, _SUB)

        @pl.when(s2 == 0)
        def _():
          drain_idx(b2)

        pltpu.async_copy(x0_h.at[colb.at[b2, s2]], rowsb.at[1 - p], gsem)

      pltpu.async_copy(rowsb.at[p], acc.at[rowb.at[bb, s]], ssem, add=True)

    drain_payload(ssem, lax.rem(_CHUNKS - 1, 2))
    plsc.subcore_barrier()
    pltpu.sync_copy(acc.at[pl.ds(sid * _RPT, _RPT)],
                    out_h.at[pl.ds(cid * _N + sid * _RPT, _RPT)])

  return k(col2, row2, val2, x0)


def _mix(partials, wbd, brow):
  bn = 2048

  def body(p_ref, w_ref, b_ref, o_ref):
    s = p_ref[0] + p_ref[1]
    zt = lax.dot_general(w_ref[:], s, (((0,), (1,)), ((), ())),
                         preferred_element_type=jnp.float32)
    o_ref[:] = (zt + b_ref[:]).reshape(2, 16, bn)

  return pl.pallas_call(
      body,
      grid=(_N // bn,),
      in_specs=[
          pl.BlockSpec((2, bn, _C), lambda i: (0, i, 0)),
          pl.BlockSpec((_C, _C), lambda i: (0, 0)),
          pl.BlockSpec((_C, 1), lambda i: (0, 0)),
      ],
      out_specs=pl.BlockSpec((2, 16, bn), lambda i: (0, 0, i)),
      out_shape=jax.ShapeDtypeStruct((2, 16, _N), jnp.float32),
  )(partials, wbd, brow)


def kernel(D_indices, D_values, x, theta, bias):
  row = D_indices[0].reshape(_NNZ // _K, _K)
  col = D_indices[1].reshape(_NNZ // _K, _K)
  val = D_values.reshape(_NNZ // _K, _K)
  x0 = jnp.transpose(x, (2, 0, 1)).reshape(_M, _C)
  partials = _sc_spmm(col, row, val, x0).reshape(_NC, _N, _C)
  wbd = jnp.kron(jnp.eye(2, dtype=theta.dtype), theta.T)
  brow = jnp.concatenate([bias[0, :, 0], bias[0, :, 0]]).reshape(_C, 1)
  return _mix(partials, wbd, brow)


# speedup vs baseline: 44.7997x; 1.7445x over previous
import functools

import jax
import jax.numpy as jnp
from jax import lax
from jax.experimental import pallas as pl
from jax.experimental.pallas import tpu as pltpu
from jax.experimental.pallas import tpu_sc as plsc

_N = 16384
_M = 16384
_NNZ = 2621440
_C = 32

_NC = 2
_NS = 16
_NW = _NC * _NS
_K = 128
_SUB = 8
_PER_W = _NNZ // _NW
_CHUNKS = _PER_W // _K
_NB = _CHUNKS // _SUB
_RPT = _N // _NS
_ZR = 128


def _sc_spmm(col2, row2, val2, x0):
  mesh = plsc.VectorSubcoreMesh(core_axis_name="c", subcore_axis_name="s")

  @functools.partial(
      pl.kernel,
      out_type=jax.ShapeDtypeStruct((_NC * _N, _C), jnp.float32),
      mesh=mesh,
      scratch_types=[
          pltpu.VMEM((2, _SUB, _K), jnp.int32),
          pltpu.VMEM((2, _SUB, _K), jnp.int32),
          pltpu.VMEM((2, _SUB, _K), jnp.float32),
          pltpu.VMEM((4, _K, _C), jnp.float32),
          pltpu.VMEM((_ZR, _C), jnp.float32),
          pltpu.VMEM_SHARED((_N, _C), jnp.float32),
          pltpu.SemaphoreType.DMA,
          pltpu.SemaphoreType.DMA,
          pltpu.SemaphoreType.DMA,
      ],
      compiler_params=pltpu.CompilerParams(use_tc_tiling_on_sc=False),
  )
  def k(col_h, row_h, val_h, x0_h, out_h,
        colb, rowb, valb, rowsb, zero_v, acc, isem, gsem, ssem):
    cid = lax.axis_index("c")
    sid = lax.axis_index("s")
    wid = sid * _NC + cid

    zeros16 = jnp.zeros((16,), jnp.float32)

    @pl.loop(0, _ZR)
    def _(r):
      zero_v[r, pl.ds(0, 16)] = zeros16
      zero_v[r, pl.ds(16, 16)] = zeros16

    for b in range(_RPT // _ZR):
      pltpu.sync_copy(zero_v, acc.at[pl.ds(sid * _RPT + b * _ZR, _ZR)])
    plsc.subcore_barrier()

    crow0 = wid * _CHUNKS

    def stage_block(b, slot):
      pltpu.async_copy(col_h.at[pl.ds(crow0 + b * _SUB, _SUB)],
                       colb.at[slot], isem)
      pltpu.async_copy(row_h.at[pl.ds(crow0 + b * _SUB, _SUB)],
                       rowb.at[slot], isem)
      pltpu.async_copy(val_h.at[pl.ds(crow0 + b * _SUB, _SUB)],
                       valb.at[slot], isem)

    def drain_idx(slot):
      pltpu.make_async_copy(col_h.at[pl.ds(crow0, _SUB)],
                            colb.at[slot], isem).wait()
      pltpu.make_async_copy(row_h.at[pl.ds(crow0, _SUB)],
                            rowb.at[slot], isem).wait()
      pltpu.make_async_copy(val_h.at[pl.ds(crow0, _SUB)],
                            valb.at[slot], isem).wait()

    def drain_payload(sem, p):
      pltpu.make_async_copy(x0_h.at[pl.ds(0, _K)], rowsb.at[p], sem).wait()

    stage_block(0, 0)
    drain_idx(0)
    pltpu.async_copy(x0_h.at[colb.at[0, 0]], rowsb.at[0], gsem)
    pltpu.async_copy(x0_h.at[colb.at[0, 1]], rowsb.at[1], gsem)

    _LOOK = 2

    @pl.loop(0, _CHUNKS)
    def _(q):
      p = lax.rem(q, 4)
      b = lax.div(q, _SUB)
      s = lax.rem(q, _SUB)
      bb = lax.rem(b, 2)

      @pl.when(jnp.logical_and(s == 0, b + 1 < _NB))
      def _():
        stage_block(b + 1, lax.rem(b + 1, 2))

      drain_payload(gsem, p)

      @pl.loop(0, _K // 16)
      def _(g):
        vvec = valb[bb, s, pl.ds(g * 16, 16)]
        for j in range(16):
          v = vvec[j]
          r = g * 16 + j
          rowsb[p, r, pl.ds(0, 16)] = rowsb[p, r, pl.ds(0, 16)] * v
          rowsb[p, r, pl.ds(16, 16)] = rowsb[p, r, pl.ds(16, 16)] * v

      @pl.when(q >= _LOOK)
      def _():
        drain_payload(ssem, lax.rem(q + _LOOK, 4))

      @pl.when(q + _LOOK < _CHUNKS)
      def _():
        q2 = q + _LOOK
        b2 = lax.rem(lax.div(q2, _SUB), 2)
        s2 = lax.rem(q2, _SUB)

        @pl.when(s2 == 0)
        def _():
          drain_idx(b2)

        pltpu.async_copy(x0_h.at[colb.at[b2, s2]], rowsb.at[lax.rem(q2, 4)],
                         gsem)

      pltpu.async_copy(rowsb.at[p], acc.at[rowb.at[bb, s]], ssem, add=True)

    drain_payload(ssem, lax.rem(_CHUNKS - 2, 4))
    drain_payload(ssem, lax.rem(_CHUNKS - 1, 4))
    plsc.subcore_barrier()
    pltpu.sync_copy(acc.at[pl.ds(sid * _RPT, _RPT)],
                    out_h.at[pl.ds(cid * _N + sid * _RPT, _RPT)])

  return k(col2, row2, val2, x0)


def _mix(partials, wbd, brow):
  bn = 2048

  def body(p_ref, w_ref, b_ref, o_ref):
    s = p_ref[0] + p_ref[1]
    zt = lax.dot_general(w_ref[:], s, (((0,), (1,)), ((), ())),
                         preferred_element_type=jnp.float32)
    o_ref[:] = (zt + b_ref[:]).reshape(2, 16, bn)

  return pl.pallas_call(
      body,
      grid=(_N // bn,),
      in_specs=[
          pl.BlockSpec((2, bn, _C), lambda i: (0, i, 0)),
          pl.BlockSpec((_C, _C), lambda i: (0, 0)),
          pl.BlockSpec((_C, 1), lambda i: (0, 0)),
      ],
      out_specs=pl.BlockSpec((2, 16, bn), lambda i: (0, 0, i)),
      out_shape=jax.ShapeDtypeStruct((2, 16, _N), jnp.float32),
  )(partials, wbd, brow)


def kernel(D_indices, D_values, x, theta, bias):
  row = D_indices[0].reshape(_NNZ // _K, _K)
  col = D_indices[1].reshape(_NNZ // _K, _K)
  val = D_values.reshape(_NNZ // _K, _K)
  x0 = jnp.transpose(x, (2, 0, 1)).reshape(_M, _C)
  partials = _sc_spmm(col, row, val, x0).reshape(_NC, _N, _C)
  wbd = jnp.kron(jnp.eye(2, dtype=theta.dtype), theta.T)
  brow = jnp.concatenate([bias[0, :, 0], bias[0, :, 0]]).reshape(_C, 1)
  return _mix(partials, wbd, brow)


# speedup vs baseline: 62.9263x; 1.4046x over previous
import functools

import jax
import jax.numpy as jnp
from jax import lax
from jax.experimental import pallas as pl
from jax.experimental.pallas import tpu as pltpu
from jax.experimental.pallas import tpu_sc as plsc

_N = 16384
_M = 16384
_NNZ = 2621440
_C = 32

_NC = 2
_NS = 16
_NW = _NC * _NS
_K = 128
_SUB = 8
_PER_W = _NNZ // _NW
_CHUNKS = _PER_W // _K
_NB = _CHUNKS // _SUB
_RPT = _N // _NS
_ZR = 128


def _sc_spmm(col2, row2, val2, x0):
  mesh = plsc.VectorSubcoreMesh(core_axis_name="c", subcore_axis_name="s")

  @functools.partial(
      pl.kernel,
      out_type=jax.ShapeDtypeStruct((_NC * _N, _C), jnp.float32),
      mesh=mesh,
      scratch_types=[
          pltpu.VMEM((2, _SUB, _K), jnp.int32),
          pltpu.VMEM((2, _SUB, _K), jnp.int32),
          pltpu.VMEM((2, _SUB, _K), jnp.float32),
          pltpu.VMEM((8, _K, _C), jnp.float32),
          pltpu.VMEM((_ZR, _C), jnp.float32),
          pltpu.VMEM_SHARED((_N, _C), jnp.float32),
          pltpu.SemaphoreType.DMA,
          pltpu.SemaphoreType.DMA,
          pltpu.SemaphoreType.DMA,
      ],
      compiler_params=pltpu.CompilerParams(use_tc_tiling_on_sc=False),
  )
  def k(col_h, row_h, val_h, x0_h, out_h,
        colb, rowb, valb, rowsb, zero_v, acc, isem, gsem, ssem):
    cid = lax.axis_index("c")
    sid = lax.axis_index("s")
    wid = sid * _NC + cid

    zeros16 = jnp.zeros((16,), jnp.float32)

    @pl.loop(0, _ZR)
    def _(r):
      zero_v[r, pl.ds(0, 16)] = zeros16
      zero_v[r, pl.ds(16, 16)] = zeros16

    for b in range(_RPT // _ZR):
      pltpu.sync_copy(zero_v, acc.at[pl.ds(sid * _RPT + b * _ZR, _ZR)])
    plsc.subcore_barrier()

    crow0 = wid * _CHUNKS

    def stage_block(b, slot):
      pltpu.async_copy(col_h.at[pl.ds(crow0 + b * _SUB, _SUB)],
                       colb.at[slot], isem)
      pltpu.async_copy(row_h.at[pl.ds(crow0 + b * _SUB, _SUB)],
                       rowb.at[slot], isem)
      pltpu.async_copy(val_h.at[pl.ds(crow0 + b * _SUB, _SUB)],
                       valb.at[slot], isem)

    def drain_idx(slot):
      pltpu.make_async_copy(col_h.at[pl.ds(crow0, _SUB)],
                            colb.at[slot], isem).wait()
      pltpu.make_async_copy(row_h.at[pl.ds(crow0, _SUB)],
                            rowb.at[slot], isem).wait()
      pltpu.make_async_copy(val_h.at[pl.ds(crow0, _SUB)],
                            valb.at[slot], isem).wait()

    def drain_payload(sem, p):
      pltpu.make_async_copy(x0_h.at[pl.ds(0, _K)], rowsb.at[p], sem).wait()

    stage_block(0, 0)
    drain_idx(0)
    for c0 in range(4):
      pltpu.async_copy(x0_h.at[colb.at[0, c0]], rowsb.at[c0], gsem)

    _LOOK = 4

    @pl.loop(0, _CHUNKS)
    def _(q):
      p = lax.rem(q, 8)
      b = lax.div(q, _SUB)
      s = lax.rem(q, _SUB)
      bb = lax.rem(b, 2)

      @pl.when(jnp.logical_and(s == 0, b + 1 < _NB))
      def _():
        stage_block(b + 1, lax.rem(b + 1, 2))

      drain_payload(gsem, p)

      @pl.loop(0, _K // 16)
      def _(g):
        vvec = valb[bb, s, pl.ds(g * 16, 16)]
        for j in range(16):
          v = vvec[j]
          r = g * 16 + j
          rowsb[p, r, pl.ds(0, 16)] = rowsb[p, r, pl.ds(0, 16)] * v
          rowsb[p, r, pl.ds(16, 16)] = rowsb[p, r, pl.ds(16, 16)] * v

      @pl.when(q >= _LOOK)
      def _():
        drain_payload(ssem, lax.rem(q + _LOOK, 8))

      @pl.when(q + _LOOK < _CHUNKS)
      def _():
        q2 = q + _LOOK
        b2 = lax.rem(lax.div(q2, _SUB), 2)
        s2 = lax.rem(q2, _SUB)

        @pl.when(s2 == 0)
        def _():
          drain_idx(b2)

        pltpu.async_copy(x0_h.at[colb.at[b2, s2]], rowsb.at[lax.rem(q2, 8)],
                         gsem)

      pltpu.async_copy(rowsb.at[p], acc.at[rowb.at[bb, s]], ssem, add=True)

    for qt in range(_CHUNKS - 4, _CHUNKS):
      drain_payload(ssem, qt % 8)
    plsc.subcore_barrier()
    pltpu.sync_copy(acc.at[pl.ds(sid * _RPT, _RPT)],
                    out_h.at[pl.ds(cid * _N + sid * _RPT, _RPT)])

  return k(col2, row2, val2, x0)


def _mix(partials, wbd, brow):
  bn = 2048

  def body(p_ref, w_ref, b_ref, o_ref):
    s = p_ref[0] + p_ref[1]
    zt = lax.dot_general(w_ref[:], s, (((0,), (1,)), ((), ())),
                         preferred_element_type=jnp.float32)
    o_ref[:] = (zt + b_ref[:]).reshape(2, 16, bn)

  return pl.pallas_call(
      body,
      grid=(_N // bn,),
      in_specs=[
          pl.BlockSpec((2, bn, _C), lambda i: (0, i, 0)),
          pl.BlockSpec((_C, _C), lambda i: (0, 0)),
          pl.BlockSpec((_C, 1), lambda i: (0, 0)),
      ],
      out_specs=pl.BlockSpec((2, 16, bn), lambda i: (0, 0, i)),
      out_shape=jax.ShapeDtypeStruct((2, 16, _N), jnp.float32),
  )(partials, wbd, brow)


def kernel(D_indices, D_values, x, theta, bias):
  row = D_indices[0].reshape(_NNZ // _K, _K)
  col = D_indices[1].reshape(_NNZ // _K, _K)
  val = D_values.reshape(_NNZ // _K, _K)
  x0 = jnp.transpose(x, (2, 0, 1)).reshape(_M, _C)
  partials = _sc_spmm(col, row, val, x0).reshape(_NC, _N, _C)
  wbd = jnp.kron(jnp.eye(2, dtype=theta.dtype), theta.T)
  brow = jnp.concatenate([bias[0, :, 0], bias[0, :, 0]]).reshape(_C, 1)
  return _mix(partials, wbd, brow)
